# baseline (speedup 1.0000x reference)
import functools

import jax
import jax.numpy as jnp
from jax import lax
from jax.experimental import pallas as pl
from jax.experimental.pallas import tpu as pltpu
from jax.experimental.pallas import tpu_sc as plsc

N = 10000
N_PAD = 10240
E = 640000
E_REAL = E + N
D_IN = 768
H = 128
NC = 2
NS = 16
NW = NC * NS
E_PAD = 655360
EPW = E_PAD // NW
CA = 2048
NCA = EPW // CA
CB = 256
NCB = EPW // CB
ROWS_PT = N_PAD // NS
FH = 32
NPASS = H // FH
GRID = 16
BR = N_PAD // GRID

_mesh = plsc.VectorSubcoreMesh(core_axis_name="c", subcore_axis_name="s")


def _tc1_body(x_ref, w_ref, a_ref, xl_ref, al_ref):
    xl = x_ref[...] @ w_ref[...]
    xl_ref[...] = xl
    al_ref[...] = xl @ a_ref[...]


def _merge_parts(p_ref):
    return jnp.concatenate(
        [p_ref[0, q] + p_ref[1, q] for q in range(NPASS)], axis=-1)


def _tc2_body(p_ref, b_ref, w_ref, a_ref, xl_ref, al_ref):
    h = _merge_parts(p_ref) + b_ref[...]
    h = jnp.where(h > 0, h, jnp.exp(jnp.minimum(h, 0.0)) - 1.0)
    xl = h @ w_ref[...]
    xl_ref[...] = xl
    al_ref[...] = xl @ a_ref[...]


def _tc3_body(p_ref, b_ref, w_ref, bp_ref, o_ref):
    h = _merge_parts(p_ref) + b_ref[...]
    o_ref[...] = jnp.maximum(h @ w_ref[...] + bp_ref[...], 0.0)


def _tc1(xp, W, A):
    return pl.pallas_call(
        _tc1_body,
        grid=(GRID,),
        in_specs=[
            pl.BlockSpec((BR, D_IN), lambda i: (i, 0)),
            pl.BlockSpec((D_IN, H), lambda i: (0, 0)),
            pl.BlockSpec((H, 2), lambda i: (0, 0)),
        ],
        out_specs=[
            pl.BlockSpec((BR, H), lambda i: (i, 0)),
            pl.BlockSpec((BR, 2), lambda i: (i, 0)),
        ],
        out_shape=[
            jax.ShapeDtypeStruct((N_PAD, H), jnp.float32),
            jax.ShapeDtypeStruct((N_PAD, 2), jnp.float32),
        ],
    )(xp, W, A)


def _tc2(parts, b, W, A):
    return pl.pallas_call(
        _tc2_body,
        grid=(GRID,),
        in_specs=[
            pl.BlockSpec((NC, NPASS, BR, FH), lambda i: (0, 0, i, 0)),
            pl.BlockSpec((1, H), lambda i: (0, 0)),
            pl.BlockSpec((H, H), lambda i: (0, 0)),
            pl.BlockSpec((H, 2), lambda i: (0, 0)),
        ],
        out_specs=[
            pl.BlockSpec((BR, H), lambda i: (i, 0)),
            pl.BlockSpec((BR, 2), lambda i: (i, 0)),
        ],
        out_shape=[
            jax.ShapeDtypeStruct((N_PAD, H), jnp.float32),
            jax.ShapeDtypeStruct((N_PAD, 2), jnp.float32),
        ],
    )(parts, b, W, A)


def _tc3(parts, b, W, bp):
    return pl.pallas_call(
        _tc3_body,
        grid=(GRID,),
        in_specs=[
            pl.BlockSpec((NC, NPASS, BR, FH), lambda i: (0, 0, i, 0)),
            pl.BlockSpec((1, H), lambda i: (0, 0)),
            pl.BlockSpec((H, H), lambda i: (0, 0)),
            pl.BlockSpec((1, H), lambda i: (0, 0)),
        ],
        out_specs=pl.BlockSpec((BR, H), lambda i: (i, 0)),
        out_shape=jax.ShapeDtypeStruct((N_PAD, H), jnp.float32),
    )(parts, b, W, bp)


@functools.partial(
    pl.kernel,
    out_type=(
        jax.ShapeDtypeStruct((E_PAD,), jnp.float32),
        jax.ShapeDtypeStruct((NC, N_PAD), jnp.float32),
    ),
    mesh=_mesh,
    compiler_params=pltpu.CompilerParams(needs_layout_passes=False, use_tc_tiling_on_sc=False),
    scratch_types=(
        pltpu.VMEM((CA,), jnp.int32),
        pltpu.VMEM((CA,), jnp.int32),
        pltpu.VMEM((CA,), jnp.float32),
        pltpu.VMEM((N_PAD,), jnp.float32),
        pltpu.VMEM((N_PAD,), jnp.float32),
        pltpu.VMEM((ROWS_PT,), jnp.float32),
        pltpu.VMEM_SHARED((N_PAD,), jnp.float32),
    ),
)
def _sc_softmax_denom(s_hbm, d_hbm, al_hbm, ex_hbm, den_hbm,
                      s_v, d_v, ex_v, as_v, ad_v, z_v, den_sh):
    cid = lax.axis_index("c")
    sid = lax.axis_index("s")
    wid = cid * NS + sid

    def zb(j, carry):
        z_v[pl.ds(j * 16, 16)] = jnp.zeros((16,), jnp.float32)
        return carry
    lax.fori_loop(0, ROWS_PT // 16, zb, 0)
    pltpu.sync_copy(z_v, den_sh.at[pl.ds(sid * ROWS_PT, ROWS_PT)])
    pltpu.sync_copy(al_hbm.at[0], as_v)
    pltpu.sync_copy(al_hbm.at[1], ad_v)
    plsc.subcore_barrier()

    def chunk_body(ci, carry):
        base = wid * EPW + ci * CA
        pltpu.sync_copy(s_hbm.at[pl.ds(base, CA)], s_v)
        pltpu.sync_copy(d_hbm.at[pl.ds(base, CA)], d_v)

        @plsc.parallel_loop(0, CA // 16, unroll=4)
        def _vec_body(k):
            sl = pl.ds(k * 16, 16)
            si = s_v[sl]
            di = d_v[sl]
            av = plsc.load_gather(as_v, [si])
            bv = plsc.load_gather(ad_v, [di])
            e = av + bv
            e = jnp.where(e > 0, e, 0.2 * e)
            exv = jnp.exp(e)
            g = base + k * 16 + lax.iota(jnp.int32, 16)
            exv = jnp.where(g < E_REAL, exv, 0.0)
            ex_v[sl] = exv
        pltpu.sync_copy(ex_v, ex_hbm.at[pl.ds(base, CA)])
        pltpu.sync_copy(ex_v, den_sh.at[d_v], add=True)
        return carry
    lax.fori_loop(0, NCA, chunk_body, 0)

    plsc.subcore_barrier()
    sl = pl.ds(sid * ROWS_PT, ROWS_PT)
    pltpu.sync_copy(den_sh.at[sl], den_hbm.at[cid, sl])


CP = 2048
NCP = EPW // CP


@functools.partial(
    pl.kernel,
    out_type=jax.ShapeDtypeStruct((NC, NPASS, N_PAD, FH), jnp.float32),
    mesh=_mesh,
    compiler_params=pltpu.CompilerParams(needs_layout_passes=False, use_tc_tiling_on_sc=False),
    scratch_types=(
        pltpu.VMEM((CP,), jnp.int32),
        pltpu.VMEM((CP,), jnp.int32),
        pltpu.VMEM((CP,), jnp.float32),
        pltpu.VMEM((2, CB), jnp.int32),
        pltpu.VMEM((2, CB), jnp.int32),
        pltpu.VMEM((EPW,), jnp.float32),
        pltpu.VMEM((EPW,), jnp.int32),
        pltpu.VMEM((EPW,), jnp.int32),
        pltpu.VMEM((N_PAD,), jnp.float32),
        pltpu.VMEM((N_PAD,), jnp.float32),
        pltpu.VMEM((2, CB, FH), jnp.float32),
        pltpu.VMEM((128, FH), jnp.float32),
        pltpu.VMEM_SHARED((N_PAD, FH), jnp.float32),
        pltpu.SemaphoreType.DMA,
        pltpu.SemaphoreType.DMA,
    ),
)
def _sc_aggregate(s_hbm, d_hbm, ex_hbm, den_hbm, xlh_hbm, out_hbm,
                  sp_v, dp_v, ep_v, dv, i2b, w_full, i2_full, d_full, den_v,
                  den2_v, rows, zrow_v, acc_sh, gsem, ssem):
    cid = lax.axis_index("c")
    sid = lax.axis_index("s")
    wid = cid * NS + sid
    ebase = wid * EPW
    zeros16 = jnp.zeros((16,), jnp.int32)

    pltpu.sync_copy(den_hbm.at[0], den_v)
    pltpu.sync_copy(den_hbm.at[1], den2_v)

    @plsc.parallel_loop(0, N_PAD // 16, unroll=4)
    def _dsum(j):
        sl = pl.ds(j * 16, 16)
        den_v[sl] = den_v[sl] + den2_v[sl] + 1e-16

    def prep(cp, carry):
        pbase = cp * CP
        pltpu.sync_copy(s_hbm.at[pl.ds(ebase + pbase, CP)], sp_v)
        pltpu.sync_copy(d_hbm.at[pl.ds(ebase + pbase, CP)], dp_v)
        pltpu.sync_copy(ex_hbm.at[pl.ds(ebase + pbase, CP)], ep_v)

        @plsc.parallel_loop(0, CP // 16, unroll=4)
        def _wk(k):
            sl = pl.ds(k * 16, 16)
            gsl = pl.ds(pbase + k * 16, 16)
            dnm = plsc.load_gather(den_v, [dp_v[sl]])
            w_full[gsl] = ep_v[sl] / dnm
            i2_full[gsl] = sp_v[sl] * NPASS
            d_full[gsl] = dp_v[sl]
        return carry
    lax.fori_loop(0, NCP, prep, 0)

    @plsc.parallel_loop(0, 128, unroll=8)
    def _zr(j):
        for f in range(FH // 16):
            zrow_v[j, pl.ds(f * 16, 16)] = jnp.zeros((16,), jnp.float32)

    for hf in range(NPASS):
        if hf >= 1:
            @plsc.parallel_loop(0, EPW // 16, unroll=4)
            def _bump(j):
                sl = pl.ds(j * 16, 16)
                i2_full[sl] = i2_full[sl] + 1

        def zacc(q, carry):
            pltpu.sync_copy(zrow_v, acc_sh.at[pl.ds(sid * ROWS_PT + q * 128, 128)])
            return carry
        lax.fori_loop(0, ROWS_PT // 128, zacc, 0)
        plsc.subcore_barrier()

        @plsc.parallel_loop(0, CB // 16, unroll=4)
        def _i2cp0(k):
            sl = pl.ds(k * 16, 16)
            i2b[0, sl] = i2_full[sl]
        pltpu.async_copy(xlh_hbm.at[i2b.at[0]], rows.at[0], gsem)

        def group(g2, carry):
            for b in range(2):
                ci = g2 * 2 + b
                nb = (b + 1) % 2
                pltpu.make_async_copy(
                    xlh_hbm.at[pl.ds(0, CB)], rows.at[b], gsem).wait()
                @pl.when(ci >= 1)
                def _():
                    pltpu.make_async_copy(
                        rows.at[nb], acc_sh.at[dv.at[nb]], ssem).wait()
                @pl.when(ci + 1 < NCB)
                def _():
                    @plsc.parallel_loop(0, CB // 16, unroll=4)
                    def _i2cp(k):
                        sl = pl.ds(k * 16, 16)
                        i2b[nb, sl] = i2_full[pl.ds((ci + 1) * CB + k * 16, 16)]
                    pltpu.async_copy(
                        xlh_hbm.at[i2b.at[nb]], rows.at[nb], gsem)
                @plsc.parallel_loop(0, CB // 16, unroll=4)
                def _dcp(k):
                    sl = pl.ds(k * 16, 16)
                    dv[b, sl] = d_full[pl.ds(ci * CB + k * 16, 16)]

                @plsc.parallel_loop(0, CB, unroll=16)
                def _scale(j):
                    w = plsc.load_gather(w_full, [zeros16 + (ci * CB + j)])
                    for f in range(FH // 16):
                        sl = pl.ds(f * 16, 16)
                        rows[b, j, sl] = rows[b, j, sl] * w

                pltpu.async_copy(rows.at[b], acc_sh.at[dv.at[b]], ssem,
                                 add=True)
            return carry
        lax.fori_loop(0, NCB // 2, group, 0)

        pltpu.make_async_copy(rows.at[1], acc_sh.at[dv.at[1]], ssem).wait()

        plsc.subcore_barrier()
        slr = pl.ds(sid * ROWS_PT, ROWS_PT)
        pltpu.sync_copy(acc_sh.at[slr], out_hbm.at[cid, hf, slr])
        plsc.subcore_barrier()


def kernel(x, edge_index, W1, a_src1, a_dst1, b1, W2, a_src2, a_dst2, b2, Wp, bp):
    src = edge_index[0]
    dst = edge_index[1]
    loop = jnp.arange(N, dtype=src.dtype)
    padi = (jnp.arange(E_PAD - E_REAL, dtype=src.dtype) * 37) % N
    s = jnp.concatenate([src, loop, padi])
    d = jnp.concatenate([dst, loop, padi])
    xp = jnp.pad(x, ((0, N_PAD - N), (0, 0)))
    A1 = jnp.stack([a_src1, a_dst1], axis=1)
    A2 = jnp.stack([a_src2, a_dst2], axis=1)

    xl1, al1 = _tc1(xp, W1, A1)
    ex1, den1 = _sc_softmax_denom(s, d, al1.T)
    parts1 = _sc_aggregate(s, d, ex1, den1, xl1.reshape(N_PAD * NPASS, FH))

    xl2, al2 = _tc2(parts1, b1.reshape(1, H), W2, A2)
    ex2, den2 = _sc_softmax_denom(s, d, al2.T)
    parts2 = _sc_aggregate(s, d, ex2, den2, xl2.reshape(N_PAD * NPASS, FH))

    out = _tc3(parts2, b2.reshape(1, H), Wp, bp.reshape(1, H))
    return out[:N]


# speedup vs baseline: 1.0032x; 1.0032x over previous
import functools

import jax
import jax.numpy as jnp
from jax import lax
from jax.experimental import pallas as pl
from jax.experimental.pallas import tpu as pltpu
from jax.experimental.pallas import tpu_sc as plsc

N = 10000
N_PAD = 10240
E = 640000
E_REAL = E + N
D_IN = 768
H = 128
NC = 2
NS = 16
NW = NC * NS
E_PAD = 655360
EPW = E_PAD // NW
CA = 2048
NCA = EPW // CA
CB = 256
NCB = EPW // CB
ROWS_PT = N_PAD // NS
FH = 32
NPASS = H // FH
GRID = 16
BR = N_PAD // GRID

_mesh = plsc.VectorSubcoreMesh(core_axis_name="c", subcore_axis_name="s")


def _tc1_body(x_ref, w_ref, a_ref, xl_ref, al_ref):
    xl = x_ref[...] @ w_ref[...]
    xl_ref[...] = xl
    al_ref[...] = xl @ a_ref[...]


def _merge_parts(p_ref):
    return jnp.concatenate(
        [p_ref[0, q] + p_ref[1, q] for q in range(NPASS)], axis=-1)


def _tc2_body(p_ref, b_ref, w_ref, a_ref, xl_ref, al_ref):
    h = _merge_parts(p_ref) + b_ref[...]
    h = jnp.where(h > 0, h, jnp.exp(jnp.minimum(h, 0.0)) - 1.0)
    xl = h @ w_ref[...]
    xl_ref[...] = xl
    al_ref[...] = xl @ a_ref[...]


def _tc3_body(p_ref, b_ref, w_ref, bp_ref, o_ref):
    h = _merge_parts(p_ref) + b_ref[...]
    o_ref[...] = jnp.maximum(h @ w_ref[...] + bp_ref[...], 0.0)


def _tc1(xp, W, A):
    return pl.pallas_call(
        _tc1_body,
        grid=(GRID,),
        in_specs=[
            pl.BlockSpec((BR, D_IN), lambda i: (i, 0)),
            pl.BlockSpec((D_IN, H), lambda i: (0, 0)),
            pl.BlockSpec((H, 2), lambda i: (0, 0)),
        ],
        out_specs=[
            pl.BlockSpec((BR, H), lambda i: (i, 0)),
            pl.BlockSpec((BR, 2), lambda i: (i, 0)),
        ],
        out_shape=[
            jax.ShapeDtypeStruct((N_PAD, H), jnp.float32),
            jax.ShapeDtypeStruct((N_PAD, 2), jnp.float32),
        ],
    )(xp, W, A)


def _tc2(parts, b, W, A):
    return pl.pallas_call(
        _tc2_body,
        grid=(GRID,),
        in_specs=[
            pl.BlockSpec((NC, NPASS, BR, FH), lambda i: (0, 0, i, 0)),
            pl.BlockSpec((1, H), lambda i: (0, 0)),
            pl.BlockSpec((H, H), lambda i: (0, 0)),
            pl.BlockSpec((H, 2), lambda i: (0, 0)),
        ],
        out_specs=[
            pl.BlockSpec((BR, H), lambda i: (i, 0)),
            pl.BlockSpec((BR, 2), lambda i: (i, 0)),
        ],
        out_shape=[
            jax.ShapeDtypeStruct((N_PAD, H), jnp.float32),
            jax.ShapeDtypeStruct((N_PAD, 2), jnp.float32),
        ],
    )(parts, b, W, A)


def _tc3(parts, b, W, bp):
    return pl.pallas_call(
        _tc3_body,
        grid=(GRID,),
        in_specs=[
            pl.BlockSpec((NC, NPASS, BR, FH), lambda i: (0, 0, i, 0)),
            pl.BlockSpec((1, H), lambda i: (0, 0)),
            pl.BlockSpec((H, H), lambda i: (0, 0)),
            pl.BlockSpec((1, H), lambda i: (0, 0)),
        ],
        out_specs=pl.BlockSpec((BR, H), lambda i: (i, 0)),
        out_shape=jax.ShapeDtypeStruct((N_PAD, H), jnp.float32),
    )(parts, b, W, bp)


@functools.partial(
    pl.kernel,
    out_type=(
        jax.ShapeDtypeStruct((E_PAD,), jnp.float32),
        jax.ShapeDtypeStruct((NC, N_PAD), jnp.float32),
    ),
    mesh=_mesh,
    compiler_params=pltpu.CompilerParams(needs_layout_passes=False, use_tc_tiling_on_sc=False),
    scratch_types=(
        pltpu.VMEM((CA,), jnp.int32),
        pltpu.VMEM((CA,), jnp.int32),
        pltpu.VMEM((CA,), jnp.float32),
        pltpu.VMEM((N_PAD,), jnp.float32),
        pltpu.VMEM((N_PAD,), jnp.float32),
        pltpu.VMEM((ROWS_PT,), jnp.float32),
        pltpu.VMEM_SHARED((N_PAD,), jnp.float32),
    ),
)
def _sc_softmax_denom(s_hbm, d_hbm, al_hbm, ex_hbm, den_hbm,
                      s_v, d_v, ex_v, as_v, ad_v, z_v, den_sh):
    cid = lax.axis_index("c")
    sid = lax.axis_index("s")
    wid = cid * NS + sid

    def zb(j, carry):
        z_v[pl.ds(j * 16, 16)] = jnp.zeros((16,), jnp.float32)
        return carry
    lax.fori_loop(0, ROWS_PT // 16, zb, 0)
    pltpu.sync_copy(z_v, den_sh.at[pl.ds(sid * ROWS_PT, ROWS_PT)])
    pltpu.sync_copy(al_hbm.at[0], as_v)
    pltpu.sync_copy(al_hbm.at[1], ad_v)
    plsc.subcore_barrier()

    def chunk_body(ci, carry):
        base = wid * EPW + ci * CA
        pltpu.sync_copy(s_hbm.at[pl.ds(base, CA)], s_v)
        pltpu.sync_copy(d_hbm.at[pl.ds(base, CA)], d_v)

        @plsc.parallel_loop(0, CA // 16, unroll=4)
        def _vec_body(k):
            sl = pl.ds(k * 16, 16)
            si = s_v[sl]
            di = d_v[sl]
            av = plsc.load_gather(as_v, [si])
            bv = plsc.load_gather(ad_v, [di])
            e = av + bv
            e = jnp.where(e > 0, e, 0.2 * e)
            exv = jnp.exp(e)
            g = base + k * 16 + lax.iota(jnp.int32, 16)
            exv = jnp.where(g < E_REAL, exv, 0.0)
            ex_v[sl] = exv
        pltpu.sync_copy(ex_v, ex_hbm.at[pl.ds(base, CA)])
        pltpu.sync_copy(ex_v, den_sh.at[d_v], add=True)
        return carry
    lax.fori_loop(0, NCA, chunk_body, 0)

    plsc.subcore_barrier()
    sl = pl.ds(sid * ROWS_PT, ROWS_PT)
    pltpu.sync_copy(den_sh.at[sl], den_hbm.at[cid, sl])


CP = 2048
NCP = EPW // CP


@functools.partial(
    pl.kernel,
    out_type=jax.ShapeDtypeStruct((NC, NPASS, N_PAD, FH), jnp.float32),
    mesh=_mesh,
    compiler_params=pltpu.CompilerParams(needs_layout_passes=False, use_tc_tiling_on_sc=False),
    scratch_types=(
        pltpu.VMEM((CP,), jnp.int32),
        pltpu.VMEM((CP,), jnp.int32),
        pltpu.VMEM((CP,), jnp.float32),
        pltpu.VMEM((2, CB), jnp.int32),
        pltpu.VMEM((2, CB), jnp.int32),
        pltpu.VMEM((EPW,), jnp.float32),
        pltpu.VMEM((EPW,), jnp.int32),
        pltpu.VMEM((EPW,), jnp.int32),
        pltpu.VMEM((N_PAD,), jnp.float32),
        pltpu.VMEM((N_PAD,), jnp.float32),
        pltpu.VMEM((2, CB, FH), jnp.float32),
        pltpu.VMEM((128, FH), jnp.float32),
        pltpu.VMEM_SHARED((N_PAD, FH), jnp.float32),
        pltpu.SemaphoreType.DMA,
        pltpu.SemaphoreType.DMA,
    ),
)
def _sc_aggregate(s_hbm, d_hbm, ex_hbm, den_hbm, xlh_hbm, out_hbm,
                  sp_v, dp_v, ep_v, dv, i2b, w_full, i2_full, d_full, den_v,
                  den2_v, rows, zrow_v, acc_sh, gsem, ssem):
    cid = lax.axis_index("c")
    sid = lax.axis_index("s")
    wid = cid * NS + sid
    ebase = wid * EPW
    zeros16 = jnp.zeros((16,), jnp.int32)

    pltpu.sync_copy(den_hbm.at[0], den_v)
    pltpu.sync_copy(den_hbm.at[1], den2_v)

    @plsc.parallel_loop(0, N_PAD // 16, unroll=4)
    def _dsum(j):
        sl = pl.ds(j * 16, 16)
        den_v[sl] = den_v[sl] + den2_v[sl] + 1e-16

    def prep(cp, carry):
        pbase = cp * CP
        pltpu.sync_copy(s_hbm.at[pl.ds(ebase + pbase, CP)], sp_v)
        pltpu.sync_copy(d_hbm.at[pl.ds(ebase + pbase, CP)], dp_v)
        pltpu.sync_copy(ex_hbm.at[pl.ds(ebase + pbase, CP)], ep_v)

        @plsc.parallel_loop(0, CP // 16, unroll=4)
        def _wk(k):
            sl = pl.ds(k * 16, 16)
            gsl = pl.ds(pbase + k * 16, 16)
            dnm = plsc.load_gather(den_v, [dp_v[sl]])
            w_full[gsl] = ep_v[sl] / dnm
            i2_full[gsl] = sp_v[sl] * NPASS
            d_full[gsl] = dp_v[sl]
        return carry
    lax.fori_loop(0, NCP, prep, 0)

    @plsc.parallel_loop(0, 128, unroll=8)
    def _zr(j):
        for f in range(FH // 16):
            zrow_v[j, pl.ds(f * 16, 16)] = jnp.zeros((16,), jnp.float32)

    for hf in range(NPASS):
        if hf >= 1:
            @plsc.parallel_loop(0, EPW // 16, unroll=4)
            def _bump(j):
                sl = pl.ds(j * 16, 16)
                i2_full[sl] = i2_full[sl] + 1

        def zacc(q, carry):
            pltpu.sync_copy(zrow_v, acc_sh.at[pl.ds(sid * ROWS_PT + q * 128, 128)])
            return carry
        lax.fori_loop(0, ROWS_PT // 128, zacc, 0)
        plsc.subcore_barrier()

        @plsc.parallel_loop(0, CB // 16, unroll=4)
        def _i2cp0(k):
            sl = pl.ds(k * 16, 16)
            i2b[0, sl] = i2_full[sl]
        pltpu.async_copy(xlh_hbm.at[i2b.at[0]], rows.at[0], gsem)

        def group(g2, carry):
            for b in range(2):
                ci = g2 * 2 + b
                nb = (b + 1) % 2
                pltpu.make_async_copy(
                    xlh_hbm.at[pl.ds(0, CB)], rows.at[b], gsem).wait()
                @pl.when(ci >= 1)
                def _():
                    pltpu.make_async_copy(
                        rows.at[nb], acc_sh.at[dv.at[nb]], ssem).wait()
                @pl.when(ci + 1 < NCB)
                def _():
                    @plsc.parallel_loop(0, CB // 16, unroll=4)
                    def _i2cp(k):
                        sl = pl.ds(k * 16, 16)
                        i2b[nb, sl] = i2_full[pl.ds((ci + 1) * CB + k * 16, 16)]
                    pltpu.async_copy(
                        xlh_hbm.at[i2b.at[nb]], rows.at[nb], gsem)
                @plsc.parallel_loop(0, CB // 16, unroll=4)
                def _dcp(k):
                    sl = pl.ds(k * 16, 16)
                    dv[b, sl] = d_full[pl.ds(ci * CB + k * 16, 16)]

                @plsc.parallel_loop(0, CB, unroll=8)
                def _scale(j):
                    w = plsc.load_gather(w_full, [zeros16 + (ci * CB + j)])
                    for f in range(FH // 16):
                        sl = pl.ds(f * 16, 16)
                        rows[b, j, sl] = rows[b, j, sl] * w

                pltpu.async_copy(rows.at[b], acc_sh.at[dv.at[b]], ssem,
                                 add=True)
            return carry
        lax.fori_loop(0, NCB // 2, group, 0)

        pltpu.make_async_copy(rows.at[1], acc_sh.at[dv.at[1]], ssem).wait()

        plsc.subcore_barrier()
        slr = pl.ds(sid * ROWS_PT, ROWS_PT)
        pltpu.sync_copy(acc_sh.at[slr], out_hbm.at[cid, hf, slr])
        plsc.subcore_barrier()


def kernel(x, edge_index, W1, a_src1, a_dst1, b1, W2, a_src2, a_dst2, b2, Wp, bp):
    src = edge_index[0]
    dst = edge_index[1]
    loop = jnp.arange(N, dtype=src.dtype)
    padi = (jnp.arange(E_PAD - E_REAL, dtype=src.dtype) * 37) % N
    s = jnp.concatenate([src, loop, padi])
    d = jnp.concatenate([dst, loop, padi])
    xp = jnp.pad(x, ((0, N_PAD - N), (0, 0)))
    A1 = jnp.stack([a_src1, a_dst1], axis=1)
    A2 = jnp.stack([a_src2, a_dst2], axis=1)

    xl1, al1 = _tc1(xp, W1, A1)
    ex1, den1 = _sc_softmax_denom(s, d, al1.T)
    parts1 = _sc_aggregate(s, d, ex1, den1, xl1.reshape(N_PAD * NPASS, FH))

    xl2, al2 = _tc2(parts1, b1.reshape(1, H), W2, A2)
    ex2, den2 = _sc_softmax_denom(s, d, al2.T)
    parts2 = _sc_aggregate(s, d, ex2, den2, xl2.reshape(N_PAD * NPASS, FH))

    out = _tc3(parts2, b2.reshape(1, H), Wp, bp.reshape(1, H))
    return out[:N]
